# Initial kernel scaffold; baseline (speedup 1.0000x reference)
import jax
import jax.numpy as jnp
from jax.experimental import pallas as pl


def kernel(x, W, b):
    raise NotImplementedError("write your pallas kernel here")


# speedup vs baseline: 5.0562x; 5.0562x over previous
import functools

import jax
import jax.numpy as jnp
from jax.experimental import pallas as pl
from jax.experimental.pallas import tpu as pltpu

CHANNELS = 768
N_CLUSTERS = 512
N_TOKENS = 32768
BT = 2048
N_BLOCKS = N_TOKENS // BT


def _cluster_body(x_ref, w_ref, b_ref, out_ref, cnt_ref):
    i = pl.program_id(0)

    @pl.when(i == 0)
    def _init():
        out_ref[...] = jnp.zeros_like(out_ref)
        cnt_ref[...] = jnp.zeros_like(cnt_ref)

    xb = x_ref[...]
    logits = (
        jnp.dot(xb, w_ref[...].T, preferred_element_type=jnp.float32)
        + b_ref[...]
    )
    idx = jnp.argmax(logits, axis=1).astype(jnp.int32)
    iota = jax.lax.broadcasted_iota(jnp.int32, (BT, N_CLUSTERS), 1)
    onehot = (iota == idx[:, None]).astype(jnp.float32)
    out_ref[...] += jnp.dot(onehot.T, xb, preferred_element_type=jnp.float32)
    cnt_ref[...] += jnp.sum(onehot, axis=0, keepdims=True)

    @pl.when(i == N_BLOCKS - 1)
    def _finalize():
        out_ref[...] = out_ref[...] / cnt_ref[...].T


@jax.jit
def kernel(x, W, b):
    out = pl.pallas_call(
        _cluster_body,
        grid=(N_BLOCKS,),
        in_specs=[
            pl.BlockSpec((BT, CHANNELS), lambda i: (i, 0)),
            pl.BlockSpec((N_CLUSTERS, CHANNELS), lambda i: (0, 0)),
            pl.BlockSpec((1, N_CLUSTERS), lambda i: (0, 0)),
        ],
        out_specs=pl.BlockSpec((N_CLUSTERS, CHANNELS), lambda i: (0, 0)),
        out_shape=jax.ShapeDtypeStruct((N_CLUSTERS, CHANNELS), jnp.float32),
        scratch_shapes=[pltpu.VMEM((1, N_CLUSTERS), jnp.float32)],
    )(x, W, b.reshape(1, N_CLUSTERS))
    return out


# speedup vs baseline: 6.5657x; 1.2985x over previous
import functools

import jax
import jax.numpy as jnp
from jax.experimental import pallas as pl
from jax.experimental.pallas import tpu as pltpu

CHANNELS = 768
N_CLUSTERS = 512
N_TOKENS = 32768
BT = 2048
N_BLOCKS = N_TOKENS // BT


def _cluster_body(x_ref, w_ref, b_ref, out_ref, cnt_ref):
    i = pl.program_id(0)

    @pl.when(i == 0)
    def _init():
        out_ref[...] = jnp.zeros_like(out_ref)
        cnt_ref[...] = jnp.zeros_like(cnt_ref)

    xb = x_ref[...]
    logits = (
        jnp.dot(xb, w_ref[...].T, preferred_element_type=jnp.float32)
        + b_ref[...]
    )
    rowmax = jnp.max(logits, axis=1, keepdims=True)
    onehot = (logits == rowmax).astype(jnp.bfloat16)
    out_ref[...] += jax.lax.dot_general(
        onehot,
        xb.astype(jnp.bfloat16),
        (((0,), (0,)), ((), ())),
        preferred_element_type=jnp.float32,
    )
    cnt_ref[...] += jnp.sum(onehot.astype(jnp.float32), axis=0, keepdims=True)

    @pl.when(i == N_BLOCKS - 1)
    def _finalize():
        out_ref[...] = out_ref[...] / cnt_ref[...].T


@jax.jit
def kernel(x, W, b):
    out = pl.pallas_call(
        _cluster_body,
        grid=(N_BLOCKS,),
        in_specs=[
            pl.BlockSpec((BT, CHANNELS), lambda i: (i, 0)),
            pl.BlockSpec((N_CLUSTERS, CHANNELS), lambda i: (0, 0)),
            pl.BlockSpec((1, N_CLUSTERS), lambda i: (0, 0)),
        ],
        out_specs=pl.BlockSpec((N_CLUSTERS, CHANNELS), lambda i: (0, 0)),
        out_shape=jax.ShapeDtypeStruct((N_CLUSTERS, CHANNELS), jnp.float32),
        scratch_shapes=[pltpu.VMEM((1, N_CLUSTERS), jnp.float32)],
    )(x, W, b.reshape(1, N_CLUSTERS))
    return out
